# Initial kernel scaffold; baseline (speedup 1.0000x reference)
import jax
import jax.numpy as jnp
from jax.experimental import pallas as pl


def kernel(X, edge_idx, C):
    raise NotImplementedError("write your pallas kernel here")


# speedup vs baseline: 9.5063x; 9.5063x over previous
import functools

import jax
import jax.numpy as jnp
from jax import lax
from jax.experimental import pallas as pl
from jax.experimental.pallas import tpu as pltpu
from jax.experimental.pallas import tpu_sc as plsc

_LANES = 16
_SQRT2 = 1.4142135381698608
_LN2 = 0.6931471805599453


def _ln1p_abs(ad):
    y = ad + 1.0
    bits = lax.bitcast_convert_type(y, jnp.int32)
    e_i = lax.shift_right_logical(bits, 23) - 127
    m = lax.bitcast_convert_type(
        (bits & 0x7FFFFF) | 0x3F800000, jnp.float32)
    big = m > _SQRT2
    m = jnp.where(big, m * 0.5, m)
    e_f = e_i.astype(jnp.float32) + jnp.where(big, 1.0, 0.0)
    s = (m - 1.0) / (m + 1.0)
    z = s * s
    ln_m = s * (2.0 + z * (0.6666666666666667 + z * (0.4 + z * 0.2857142857142857)))
    return e_f * _LN2 + ln_m


def _make_sc_kernel(N, K, E, n_tiles, block):
    ept = E // n_tiles
    n_blk = ept // block
    vecs = block // _LANES
    kshift = K.bit_length() - 1 if (K & (K - 1)) == 0 else None

    mesh = plsc.VectorSubcoreMesh(core_axis_name="c", subcore_axis_name="s")
    nc = mesh.num_cores

    @functools.partial(
        pl.kernel,
        out_type=jax.ShapeDtypeStruct((E * 3,), jnp.float32),
        mesh=mesh,
        compiler_params=pltpu.CompilerParams(needs_layout_passes=False),
        scratch_types=[
            pltpu.VMEM((N,), jnp.int32),
            pltpu.VMEM((block,), jnp.int32),
            pltpu.VMEM((block * 3,), jnp.float32),
        ],
    )
    def sc_kernel(c_hbm, e_hbm, out_hbm, c_v, e_v, o_v):
        wid = lax.axis_index("s") * nc + lax.axis_index("c")
        base = wid * ept
        pltpu.sync_copy(c_hbm, c_v)
        lane = lax.iota(jnp.int32, _LANES)

        def do_block(b, carry):
            off = base + b * block
            pltpu.sync_copy(e_hbm.at[pl.ds(off, block)], e_v)

            def do_vec(g, carry2):
                jv = e_v[pl.ds(g * _LANES, _LANES)]
                ev = lane + (off + g * _LANES)
                if kshift is not None:
                    iv = lax.shift_right_logical(ev, kshift)
                else:
                    iv = ev // K
                cj = plsc.load_gather(c_v, [jv])
                ci = plsc.load_gather(c_v, [iv])
                d = (jv - iv).astype(jnp.float32)
                is_if = jnp.where(ci != cj, 1.0, 0.0).astype(jnp.float32)
                intra = 1.0 - is_if
                lg = _ln1p_abs(jnp.abs(d))
                lidx = (lane + g * _LANES) * 3
                plsc.store_scatter(o_v, [lidx], is_if)
                plsc.store_scatter(o_v, [lidx + 1], intra * lg)
                plsc.store_scatter(o_v, [lidx + 2], intra * jnp.sign(d))
                return carry2

            lax.fori_loop(0, vecs, do_vec, 0, unroll=False)
            pltpu.sync_copy(o_v, out_hbm.at[pl.ds(off * 3, block * 3)])
            return carry

        lax.fori_loop(0, n_blk, do_block, 0, unroll=False)

    return sc_kernel


def kernel(X, edge_idx, C):
    B, N, K = edge_idx.shape
    E = B * N * K
    n_tiles = 32
    block = 2000
    assert E % n_tiles == 0 and (E // n_tiles) % block == 0
    c_flat = C.reshape(-1)
    e_flat = edge_idx.reshape(-1)
    out = _make_sc_kernel(N, K, E, n_tiles, block)(c_flat, e_flat)
    return out.reshape(B, N, K, 3)


# speedup vs baseline: 9.5292x; 1.0024x over previous
import functools

import jax
import jax.numpy as jnp
from jax import lax
from jax.experimental import pallas as pl
from jax.experimental.pallas import tpu as pltpu
from jax.experimental.pallas import tpu_sc as plsc

_LANES = 16
_SQRT2 = 1.4142135381698608
_LN2 = 0.6931471805599453


def _ln1p_abs(ad):
    y = ad + 1.0
    bits = lax.bitcast_convert_type(y, jnp.int32)
    e_i = lax.shift_right_logical(bits, 23) - 127
    m = lax.bitcast_convert_type(
        (bits & 0x7FFFFF) | 0x3F800000, jnp.float32)
    big = m > _SQRT2
    m = jnp.where(big, m * 0.5, m)
    e_f = e_i.astype(jnp.float32) + jnp.where(big, 1.0, 0.0)
    s = (m - 1.0) / (m + 1.0)
    z = s * s
    ln_m = s * (2.0 + z * (0.6666666666666667 + z * (0.4 + z * 0.2857142857142857)))
    return e_f * _LN2 + ln_m


def _make_sc_kernel(N, K, E, n_tiles, block):
    ept = E // n_tiles
    n_blk = ept // block
    vecs = block // _LANES
    kshift = K.bit_length() - 1 if (K & (K - 1)) == 0 else None

    mesh = plsc.VectorSubcoreMesh(core_axis_name="c", subcore_axis_name="s")
    nc = mesh.num_cores

    @functools.partial(
        pl.kernel,
        out_type=jax.ShapeDtypeStruct((E * 3,), jnp.float32),
        mesh=mesh,
        compiler_params=pltpu.CompilerParams(needs_layout_passes=False),
        scratch_types=[
            pltpu.VMEM((N + _LANES,), jnp.int32),
            pltpu.VMEM((block,), jnp.int32),
            pltpu.VMEM((block * 3,), jnp.float32),
        ],
    )
    def sc_kernel(c_hbm, e_hbm, out_hbm, c_v, e_v, o_v):
        wid = lax.axis_index("s") * nc + lax.axis_index("c")
        base = wid * ept
        pltpu.sync_copy(c_hbm, c_v.at[pl.ds(0, N)])
        lane = lax.iota(jnp.int32, _LANES)

        def do_block(b, carry):
            off = base + b * block
            pltpu.sync_copy(e_hbm.at[pl.ds(off, block)], e_v)

            def do_vec(g, carry2):
                e0 = off + g * _LANES
                if kshift is not None:
                    i0 = lax.shift_right_logical(e0, kshift)
                else:
                    i0 = e0 // K
                jv = e_v[pl.ds(g * _LANES, _LANES)]
                cj = plsc.load_gather(c_v, [jv])
                ci = c_v[pl.ds(i0, _LANES)][0]
                d = (jv - i0).astype(jnp.float32)
                is_if = jnp.where(ci != cj, 1.0, 0.0).astype(jnp.float32)
                intra = 1.0 - is_if
                lg = _ln1p_abs(jnp.abs(d))
                lidx = (lane + g * _LANES) * 3
                plsc.store_scatter(o_v, [lidx], is_if)
                plsc.store_scatter(o_v, [lidx + 1], intra * lg)
                plsc.store_scatter(o_v, [lidx + 2], intra * jnp.sign(d))
                return carry2

            lax.fori_loop(0, vecs, do_vec, 0, unroll=5)
            pltpu.sync_copy(o_v, out_hbm.at[pl.ds(off * 3, block * 3)])
            return carry

        lax.fori_loop(0, n_blk, do_block, 0, unroll=False)

    return sc_kernel


def kernel(X, edge_idx, C):
    B, N, K = edge_idx.shape
    E = B * N * K
    n_tiles = 32
    block = 2000
    assert E % n_tiles == 0 and (E // n_tiles) % block == 0
    assert K % _LANES == 0
    c_flat = C.reshape(-1)
    e_flat = edge_idx.reshape(-1)
    out = _make_sc_kernel(N, K, E, n_tiles, block)(c_flat, e_flat)
    return out.reshape(B, N, K, 3)


# speedup vs baseline: 128.2259x; 13.4561x over previous
import functools

import jax
import jax.numpy as jnp
from jax import lax
from jax.experimental import pallas as pl
from jax.experimental.pallas import tpu as pltpu
from jax.experimental.pallas import tpu_sc as plsc

_LANES = 16
_SQRT2 = 1.4142135381698608
_LN2 = 0.6931471805599453


def _ln1p_abs(ad):
    y = ad + 1.0
    bits = lax.bitcast_convert_type(y, jnp.int32)
    e_i = lax.shift_right_logical(bits, 23) - 127
    m = lax.bitcast_convert_type(
        (bits & 0x7FFFFF) | 0x3F800000, jnp.float32)
    big = m > _SQRT2
    m = jnp.where(big, m * 0.5, m)
    e_f = e_i.astype(jnp.float32) + jnp.where(big, 1.0, 0.0)
    s = (m - 1.0) / (m + 1.0)
    z = s * s
    ln_m = s * (2.0 + z * (0.6666666666666667 + z * (0.4 + z * 0.2857142857142857)))
    return e_f * _LN2 + ln_m


def _make_sc_kernel(N, K, n_main):
    KG = 4
    NS = 8
    ROWS = K // KG
    W = 896
    stripe = -(-n_main // (NS * 128)) * 128
    nblk = stripe // W
    assert stripe % W == 0
    last_len = n_main - (NS - 1) * stripe
    n_full_last = last_len // W
    tail_w = last_len - n_full_last * W
    assert tail_w % 128 == 0 and tail_w >= 0

    mesh = plsc.VectorSubcoreMesh(core_axis_name="c", subcore_axis_name="s")
    nc = mesh.num_cores

    @functools.partial(
        pl.kernel,
        out_type=jax.ShapeDtypeStruct((3, KG, ROWS, N), jnp.float32),
        mesh=mesh,
        compiler_params=pltpu.CompilerParams(needs_layout_passes=False),
        scratch_types=[
            pltpu.VMEM((N,), jnp.int32),
            pltpu.VMEM((ROWS, W), jnp.int32),
            pltpu.VMEM((ROWS, W), jnp.float32),
            pltpu.VMEM((ROWS, W), jnp.float32),
            pltpu.VMEM((ROWS, W), jnp.float32),
        ],
    )
    def sc_kernel(c_hbm, e_hbm, out_hbm, c_v, e_v, o0_v, o1_v, o2_v):
        wid = lax.axis_index("s") * nc + lax.axis_index("c")
        kg = wid % KG
        s = wid // KG
        s_lo = s * stripe
        pltpu.sync_copy(c_hbm.at[0], c_v)
        lane = lax.iota(jnp.int32, _LANES)

        def run_block(n0, width):
            n0 = pl.multiple_of(n0, 128)
            nv = width // _LANES
            pltpu.sync_copy(e_hbm.at[kg, :, pl.ds(n0, width)],
                            e_v.at[:, pl.ds(0, width)])

            def do_col(v, carry):
                col = v * _LANES
                ci = c_v[pl.ds(n0 + col, _LANES)]
                iv = n0 + col + lane
                for r in range(ROWS):
                    jv = e_v[r, pl.ds(col, _LANES)]
                    cj = plsc.load_gather(c_v, [jv])
                    d = (jv - iv).astype(jnp.float32)
                    is_if = jnp.where(ci != cj, 1.0, 0.0).astype(jnp.float32)
                    intra = 1.0 - is_if
                    lg = _ln1p_abs(jnp.abs(d))
                    o0_v[r, pl.ds(col, _LANES)] = is_if
                    o1_v[r, pl.ds(col, _LANES)] = intra * lg
                    o2_v[r, pl.ds(col, _LANES)] = intra * jnp.sign(d)
                return carry

            lax.fori_loop(0, nv, do_col, 0, unroll=False)
            for c, o_v in ((0, o0_v), (1, o1_v), (2, o2_v)):
                pltpu.sync_copy(o_v.at[:, pl.ds(0, width)],
                                out_hbm.at[c, kg, :, pl.ds(n0, width)])

        def do_block(b, carry):
            @pl.when(jnp.logical_or(s < NS - 1, b < n_full_last))
            def _():
                run_block(s_lo + b * W, W)
            return carry

        lax.fori_loop(0, nblk, do_block, 0, unroll=False)

        if tail_w:
            @pl.when(s == NS - 1)
            def _():
                run_block(s_lo + n_full_last * W, tail_w)

    return sc_kernel


def kernel(X, edge_idx, C):
    B, N, K = edge_idx.shape
    assert B == 1 and K % 4 == 0 and (K // 4) % 8 == 0
    n_main = (N // 128) * 128
    e_t = jnp.transpose(edge_idx, (0, 2, 1)).reshape(4, K // 4, N)
    out = _make_sc_kernel(N, K, n_main)(C, e_t)
    if n_main < N:
        nt = N - n_main
        it = jnp.arange(n_main, N, dtype=jnp.int32).reshape(1, nt, 1)
        ej = lax.dynamic_slice_in_dim(edge_idx, n_main, nt, axis=1)
        ci = lax.dynamic_slice_in_dim(C, n_main, nt, axis=1)[:, :, None]
        cj = jnp.take_along_axis(C[:, None, :], ej, axis=2)
        is_if = jnp.not_equal(ci, cj).astype(jnp.float32)
        dsgn = (ej - it).astype(jnp.float32)
        intra = 1.0 - is_if
        upd = jnp.stack(
            [is_if, intra * jnp.log(jnp.abs(dsgn) + 1.0),
             intra * jnp.sign(dsgn)], axis=1)
        upd = jnp.transpose(upd, (0, 1, 3, 2))
        out = lax.dynamic_update_slice(
            out.reshape(1, 3, K, N), upd, (0, 0, 0, n_main))
        out = out.reshape(3, K, N)
    return jnp.transpose(out.reshape(1, 3, K, N), (0, 3, 2, 1))


# speedup vs baseline: 140.8129x; 1.0982x over previous
import functools

import jax
import jax.numpy as jnp
from jax import lax
from jax.experimental import pallas as pl
from jax.experimental.pallas import tpu as pltpu
from jax.experimental.pallas import tpu_sc as plsc

_LANES = 16
_SQRT2 = 1.4142135381698608
_LN2 = 0.6931471805599453


def _ln1p_abs(ad):
    y = ad + 1.0
    bits = lax.bitcast_convert_type(y, jnp.int32)
    e_i = lax.shift_right_logical(bits, 23) - 127
    m = lax.bitcast_convert_type(
        (bits & 0x7FFFFF) | 0x3F800000, jnp.float32)
    big = m > _SQRT2
    m = jnp.where(big, m * 0.5, m)
    e_f = e_i.astype(jnp.float32) + jnp.where(big, 1.0, 0.0)
    s = (m - 1.0) / (m + 1.0)
    z = s * s
    ln_m = s * (2.0 + z * (0.6666666666666667 + z * (0.4 + z * 0.2857142857142857)))
    return e_f * _LN2 + ln_m


def _make_sc_kernel(N, K):
    KG = 4
    NS = 8
    ROWS = K // KG
    W = 896
    n_phys = -(-N // 128) * 128
    stripe = -(-n_phys // (NS * 128)) * 128
    nblk = stripe // W
    assert stripe % W == 0
    last_len = n_phys - (NS - 1) * stripe
    n_full_last = last_len // W
    tail_w = last_len - n_full_last * W
    assert tail_w % 128 == 0 and tail_w >= 0

    mesh = plsc.VectorSubcoreMesh(core_axis_name="c", subcore_axis_name="s")
    nc = mesh.num_cores

    @functools.partial(
        pl.kernel,
        out_type=jax.ShapeDtypeStruct((3, KG, ROWS, N), jnp.float32),
        mesh=mesh,
        compiler_params=pltpu.CompilerParams(needs_layout_passes=False),
        scratch_types=[
            pltpu.VMEM((n_phys,), jnp.int32),
            pltpu.VMEM((ROWS, W), jnp.int32),
            pltpu.VMEM((ROWS, W), jnp.float32),
            pltpu.VMEM((ROWS, W), jnp.float32),
            pltpu.VMEM((ROWS, W), jnp.float32),
        ],
    )
    def sc_kernel(c_hbm, e_hbm, out_hbm, c_v, e_v, o0_v, o1_v, o2_v):
        wid = lax.axis_index("s") * nc + lax.axis_index("c")
        kg = wid % KG
        s = wid // KG
        s_lo = s * stripe
        pltpu.sync_copy(c_hbm.at[0], c_v.at[pl.ds(0, N)])
        lane = lax.iota(jnp.int32, _LANES)

        def run_block(n0, width, clamp):
            n0 = pl.multiple_of(n0, 128)
            nv = width // _LANES
            pltpu.sync_copy(e_hbm.at[kg, :, pl.ds(n0, width)],
                            e_v.at[:, pl.ds(0, width)])

            def do_col(v, carry):
                col = v * _LANES
                ci = c_v[pl.ds(n0 + col, _LANES)]
                iv = n0 + col + lane
                for r in range(ROWS):
                    jv = e_v[r, pl.ds(col, _LANES)]
                    if clamp:
                        jv = jnp.minimum(jnp.maximum(jv, 0), N - 1)
                    cj = plsc.load_gather(c_v, [jv])
                    d = (jv - iv).astype(jnp.float32)
                    is_if = jnp.where(ci != cj, 1.0, 0.0).astype(jnp.float32)
                    intra = 1.0 - is_if
                    lg = _ln1p_abs(jnp.abs(d))
                    o0_v[r, pl.ds(col, _LANES)] = is_if
                    o1_v[r, pl.ds(col, _LANES)] = intra * lg
                    o2_v[r, pl.ds(col, _LANES)] = intra * jnp.sign(d)
                return carry

            lax.fori_loop(0, nv, do_col, 0, unroll=False)
            for c, o_v in ((0, o0_v), (1, o1_v), (2, o2_v)):
                pltpu.sync_copy(o_v.at[:, pl.ds(0, width)],
                                out_hbm.at[c, kg, :, pl.ds(n0, width)])

        def do_block(b, carry):
            @pl.when(jnp.logical_or(s < NS - 1, b < n_full_last))
            def _():
                run_block(s_lo + b * W, W, clamp=False)
            return carry

        lax.fori_loop(0, nblk, do_block, 0, unroll=False)

        if tail_w:
            @pl.when(s == NS - 1)
            def _():
                run_block(s_lo + n_full_last * W, tail_w, clamp=True)

    return sc_kernel


def kernel(X, edge_idx, C):
    B, N, K = edge_idx.shape
    assert B == 1 and K % 4 == 0 and (K // 4) % 8 == 0 and N % _LANES == 0
    e_t = jnp.transpose(edge_idx, (0, 2, 1)).reshape(4, K // 4, N)
    out = _make_sc_kernel(N, K)(C, e_t)
    return jnp.transpose(out.reshape(1, 3, K, N), (0, 3, 2, 1))
